# Initial kernel scaffold; baseline (speedup 1.0000x reference)
import jax
import jax.numpy as jnp
from jax.experimental import pallas as pl


def kernel(x, batch, W, b):
    raise NotImplementedError("write your pallas kernel here")


# speedup vs baseline: 5.9232x; 5.9232x over previous
import functools

import jax
import jax.numpy as jnp
from jax import lax
from jax.experimental import pallas as pl
from jax.experimental.pallas import tpu as pltpu
from jax.experimental.pallas import tpu_sc as plsc

N = 50000
D = 256
NSEG = 128
NCORES = 2
NSUB = 16
NW = NCORES * NSUB
SEG_PER_W = NSEG // NW
CH = 128
NG = D // 16

_NEG_INF = float("-inf")
_POS_INF = float("inf")


def _pool_kernel(x_hbm, batch_hbm, out_hbm, batch_v, xbuf, accbuf, sem):
    wid = lax.axis_index("s") * NCORES + lax.axis_index("c")
    s_base = wid * SEG_PER_W

    pltpu.sync_copy(batch_hbm, batch_v.at[pl.ds(0, N)])
    batch_v[pl.ds(N, 16)] = jnp.full((16,), NSEG, jnp.int32)

    def lower_bound(target):
        def body(_, lohi):
            lo, hi = lohi
            mid = (lo + hi) // 2
            v = batch_v[pl.ds(mid, 16)][0]
            pred = v < target
            return jnp.where(pred, mid + 1, lo), jnp.where(pred, hi, mid)

        lo, _ = lax.fori_loop(0, 16, body, (jnp.int32(0), jnp.int32(N)))
        return lo

    offs = [lower_bound(s_base + k) for k in range(SEG_PER_W + 1)]

    for k in range(SEG_PER_W):
        seg = s_base + k
        o_s = offs[k]
        o_e = offs[k + 1]
        cnt = o_e - o_s

        for g in range(NG):
            accbuf[pl.ds(g * 16, 16)] = jnp.full((16,), _NEG_INF, jnp.float32)
            accbuf[pl.ds(D + g * 16, 16)] = jnp.full((16,), _POS_INF, jnp.float32)
            accbuf[pl.ds(2 * D + g * 16, 16)] = jnp.zeros((16,), jnp.float32)

        nchunks = (cnt + CH - 1) // CH

        def chunk_body(i, _):
            st = o_s + i * CH
            n = jnp.minimum(CH, o_e - st)
            std = jnp.minimum(st, N - CH)
            d = st - std
            pltpu.sync_copy(x_hbm.at[pl.ds(std * D, CH * D)], xbuf)
            for g in range(NG):
                mx0 = accbuf[pl.ds(g * 16, 16)]
                mn0 = accbuf[pl.ds(D + g * 16, 16)]
                sm0 = accbuf[pl.ds(2 * D + g * 16, 16)]

                def row_body(r, carry):
                    mx, mn, sm = carry
                    v = xbuf[pl.ds(r * D + g * 16, 16)]
                    return (jnp.maximum(mx, v), jnp.minimum(mn, v), sm + v)

                mx1, mn1, sm1 = lax.fori_loop(d, d + n, row_body, (mx0, mn0, sm0))
                accbuf[pl.ds(g * 16, 16)] = mx1
                accbuf[pl.ds(D + g * 16, 16)] = mn1
                accbuf[pl.ds(2 * D + g * 16, 16)] = sm1
            return 0

        lax.fori_loop(0, nchunks, chunk_body, 0)

        denom = jnp.maximum(cnt.astype(jnp.float32), 1.0)
        for g in range(NG):
            sm = accbuf[pl.ds(2 * D + g * 16, 16)]
            accbuf[pl.ds(2 * D + g * 16, 16)] = sm / denom

        pltpu.sync_copy(accbuf, out_hbm.at[seg])


def _pool(x_flat, batch):
    mesh = plsc.VectorSubcoreMesh(core_axis_name="c", subcore_axis_name="s")
    call = functools.partial(
        pl.kernel,
        mesh=mesh,
        out_type=jax.ShapeDtypeStruct((NSEG, 3 * D), jnp.float32),
        scratch_types=[
            pltpu.VMEM((N + 16,), jnp.int32),
            pltpu.VMEM((CH * D,), jnp.float32),
            pltpu.VMEM((3 * D,), jnp.float32),
            pltpu.SemaphoreType.DMA,
        ],
    )(_pool_kernel)
    return call(x_flat, batch)


def _mm_kernel(feat_ref, w_ref, b_ref, out_ref):
    out_ref[...] = (
        jnp.dot(feat_ref[...], w_ref[...], preferred_element_type=jnp.float32)
        + b_ref[...]
    )


def _mm(feat, W, b):
    return pl.pallas_call(
        _mm_kernel,
        out_shape=jax.ShapeDtypeStruct((NSEG, D), jnp.float32),
    )(feat, W, b.reshape(1, D))


def kernel(x, batch, W, b):
    feat = _pool(x.reshape(-1), batch.astype(jnp.int32))
    return _mm(feat, W, b)


# speedup vs baseline: 10.3328x; 1.7444x over previous
import functools

import jax
import jax.numpy as jnp
from jax import lax
from jax.experimental import pallas as pl
from jax.experimental.pallas import tpu as pltpu
from jax.experimental.pallas import tpu_sc as plsc

N = 50000
D = 256
NSEG = 128
NCORES = 2
NSUB = 16
NW = NCORES * NSUB
SEG_PER_W = NSEG // NW
CH = 128
NG = D // 16
U = 8

_NEG_INF = float("-inf")
_POS_INF = float("inf")


def _pool_kernel(x_hbm, batch_hbm, out_hbm, batch_v, xbuf, accbuf, offs_s, sem):
    wid = lax.axis_index("s") * NCORES + lax.axis_index("c")
    s_base = wid * SEG_PER_W

    pltpu.sync_copy(batch_hbm, batch_v.at[pl.ds(0, N)])
    batch_v[pl.ds(N, 16)] = jnp.full((16,), NSEG, jnp.int32)

    def lower_bound(target):
        def body(_, lohi):
            lo, hi = lohi
            mid = (lo + hi) // 2
            v = batch_v[pl.ds(mid, 16)][0]
            pred = v < target
            return jnp.where(pred, mid + 1, lo), jnp.where(pred, hi, mid)

        lo, _ = lax.fori_loop(0, 16, body, (jnp.int32(0), jnp.int32(N)))
        return lo

    for k in range(SEG_PER_W + 1):
        offs_s[k] = lower_bound(s_base + k)

    def acc_rows(base, nrows):
        for g in range(NG):
            mx0 = accbuf[pl.ds(g * 16, 16)]
            mn0 = accbuf[pl.ds(D + g * 16, 16)]
            sm0 = accbuf[pl.ds(2 * D + g * 16, 16)]
            ninf = jnp.full((16,), _NEG_INF, jnp.float32)
            pinf = jnp.full((16,), _POS_INF, jnp.float32)
            zero = jnp.zeros((16,), jnp.float32)
            init = (mx0, ninf, ninf, ninf, mn0, pinf, pinf, pinf,
                    sm0, zero, zero, zero)

            def bodyU(j, c):
                r = base + j * U
                v = [xbuf[pl.ds((r + t) * D + g * 16, 16)] for t in range(U)]
                mx = [jnp.maximum(jnp.maximum(c[t], v[t]), v[t + 4])
                      for t in range(4)]
                mn = [jnp.minimum(jnp.minimum(c[4 + t], v[t]), v[t + 4])
                      for t in range(4)]
                sm = [c[8 + t] + v[t] + v[t + 4] for t in range(4)]
                return tuple(mx + mn + sm)

            nU = nrows // U
            c = lax.fori_loop(0, nU, bodyU, init)

            def body1(r, c3):
                mx, mn, sm = c3
                v = xbuf[pl.ds(r * D + g * 16, 16)]
                return (jnp.maximum(mx, v), jnp.minimum(mn, v), sm + v)

            mx = jnp.maximum(jnp.maximum(c[0], c[1]), jnp.maximum(c[2], c[3]))
            mn = jnp.minimum(jnp.minimum(c[4], c[5]), jnp.minimum(c[6], c[7]))
            sm = (c[8] + c[9]) + (c[10] + c[11])
            mx, mn, sm = lax.fori_loop(base + nU * U, base + nrows, body1,
                                       (mx, mn, sm))
            accbuf[pl.ds(g * 16, 16)] = mx
            accbuf[pl.ds(D + g * 16, 16)] = mn
            accbuf[pl.ds(2 * D + g * 16, 16)] = sm

    def seg_body(k, _):
        seg = s_base + k
        o_s = offs_s[k]
        o_e = offs_s[k + 1]
        cnt = o_e - o_s

        for g in range(NG):
            accbuf[pl.ds(g * 16, 16)] = jnp.full((16,), _NEG_INF, jnp.float32)
            accbuf[pl.ds(D + g * 16, 16)] = jnp.full((16,), _POS_INF, jnp.float32)
            accbuf[pl.ds(2 * D + g * 16, 16)] = jnp.zeros((16,), jnp.float32)

        nfull = cnt // CH
        rem = cnt - nfull * CH

        def chunk_body(i, _):
            st = o_s + i * CH
            pltpu.sync_copy(x_hbm.at[pl.ds(st * D, CH * D)], xbuf)
            acc_rows(jnp.int32(0), jnp.int32(CH))
            return 0

        lax.fori_loop(0, nfull, chunk_body, 0)

        @pl.when(rem > 0)
        def _():
            std = jnp.maximum(o_e - CH, 0)
            d = o_e - rem - std
            pltpu.sync_copy(x_hbm.at[pl.ds(std * D, CH * D)], xbuf)
            acc_rows(d, rem)

        denom = jnp.maximum(cnt.astype(jnp.float32), 1.0)
        for g in range(NG):
            sm = accbuf[pl.ds(2 * D + g * 16, 16)]
            accbuf[pl.ds(2 * D + g * 16, 16)] = sm / denom

        pltpu.sync_copy(accbuf, out_hbm.at[seg])
        return 0

    lax.fori_loop(0, SEG_PER_W, seg_body, 0)


def _pool(x_flat, batch):
    mesh = plsc.VectorSubcoreMesh(core_axis_name="c", subcore_axis_name="s")
    call = functools.partial(
        pl.kernel,
        mesh=mesh,
        out_type=jax.ShapeDtypeStruct((NSEG, 3 * D), jnp.float32),
        scratch_types=[
            pltpu.VMEM((N + 16,), jnp.int32),
            pltpu.VMEM((CH * D,), jnp.float32),
            pltpu.VMEM((3 * D,), jnp.float32),
            pltpu.SMEM((SEG_PER_W + 1,), jnp.int32),
            pltpu.SemaphoreType.DMA,
        ],
    )(_pool_kernel)
    return call(x_flat, batch)


def _mm_kernel(feat_ref, w_ref, b_ref, out_ref):
    out_ref[...] = (
        jnp.dot(feat_ref[...], w_ref[...], preferred_element_type=jnp.float32)
        + b_ref[...]
    )


def _mm(feat, W, b):
    return pl.pallas_call(
        _mm_kernel,
        out_shape=jax.ShapeDtypeStruct((NSEG, D), jnp.float32),
    )(feat, W, b.reshape(1, D))


def kernel(x, batch, W, b):
    feat = _pool(x.reshape(-1), batch.astype(jnp.int32))
    return _mm(feat, W, b)


# speedup vs baseline: 13.5013x; 1.3067x over previous
import functools

import jax
import jax.numpy as jnp
from jax import lax
from jax.experimental import pallas as pl
from jax.experimental.pallas import tpu as pltpu
from jax.experimental.pallas import tpu_sc as plsc

N = 50000
D = 256
NSEG = 128
NCORES = 2
NSUB = 16
NW = NCORES * NSUB
SEG_PER_W = NSEG // NW
CH = 128
NG = D // 16
U = 8

_NEG_INF = float("-inf")
_POS_INF = float("inf")


def _pool_kernel(x_hbm, batch_hbm, out_hbm, batch_v, xbuf, accbuf, offs_s, sem):
    wid = lax.axis_index("s") * NCORES + lax.axis_index("c")
    s_base = wid * SEG_PER_W

    pltpu.sync_copy(batch_hbm, batch_v.at[pl.ds(0, N)])
    batch_v[pl.ds(N, 16)] = jnp.full((16,), NSEG, jnp.int32)

    def lower_bound(target):
        def body(_, lohi):
            lo, hi = lohi
            mid = (lo + hi) // 2
            v = batch_v[pl.ds(mid, 16)][0]
            pred = v < target
            return jnp.where(pred, mid + 1, lo), jnp.where(pred, hi, mid)

        lo, _ = lax.fori_loop(0, 16, body, (jnp.int32(0), jnp.int32(N)))
        return lo

    for k in range(SEG_PER_W + 1):
        offs_s[k] = lower_bound(s_base + k)

    def acc_rows(base, nrows):
        for g in range(NG):
            mx0 = accbuf[pl.ds(g * 16, 16)]
            mn0 = accbuf[pl.ds(D + g * 16, 16)]
            sm0 = accbuf[pl.ds(2 * D + g * 16, 16)]
            ninf = jnp.full((16,), _NEG_INF, jnp.float32)
            pinf = jnp.full((16,), _POS_INF, jnp.float32)
            zero = jnp.zeros((16,), jnp.float32)
            init = (mx0, ninf, ninf, ninf, mn0, pinf, pinf, pinf,
                    sm0, zero, zero, zero)

            def bodyU(j, c):
                r = base + j * U
                v = [xbuf[r + t, pl.ds(g * 16, 16)] for t in range(U)]
                mx = [jnp.maximum(jnp.maximum(c[t], v[t]), v[t + 4])
                      for t in range(4)]
                mn = [jnp.minimum(jnp.minimum(c[4 + t], v[t]), v[t + 4])
                      for t in range(4)]
                sm = [c[8 + t] + v[t] + v[t + 4] for t in range(4)]
                return tuple(mx + mn + sm)

            nU = nrows // U
            c = lax.fori_loop(0, nU, bodyU, init)

            def body1(r, c3):
                mx, mn, sm = c3
                v = xbuf[r, pl.ds(g * 16, 16)]
                return (jnp.maximum(mx, v), jnp.minimum(mn, v), sm + v)

            mx = jnp.maximum(jnp.maximum(c[0], c[1]), jnp.maximum(c[2], c[3]))
            mn = jnp.minimum(jnp.minimum(c[4], c[5]), jnp.minimum(c[6], c[7]))
            sm = (c[8] + c[9]) + (c[10] + c[11])
            mx, mn, sm = lax.fori_loop(base + nU * U, base + nrows, body1,
                                       (mx, mn, sm))
            accbuf[pl.ds(g * 16, 16)] = mx
            accbuf[pl.ds(D + g * 16, 16)] = mn
            accbuf[pl.ds(2 * D + g * 16, 16)] = sm

    def seg_body(k, _):
        seg = s_base + k
        o_s = offs_s[k]
        o_e = offs_s[k + 1]
        cnt = o_e - o_s

        for g in range(NG):
            accbuf[pl.ds(g * 16, 16)] = jnp.full((16,), _NEG_INF, jnp.float32)
            accbuf[pl.ds(D + g * 16, 16)] = jnp.full((16,), _POS_INF, jnp.float32)
            accbuf[pl.ds(2 * D + g * 16, 16)] = jnp.zeros((16,), jnp.float32)

        a_s = (o_s // 8) * 8
        nch = (o_e - a_s + CH - 1) // CH

        def chunk_body(i, _):
            st = a_s + i * CH
            std = pl.multiple_of(jnp.minimum(st, N - CH), 8)
            pltpu.sync_copy(x_hbm.at[pl.ds(std, CH)], xbuf)
            lo = jnp.maximum(o_s, st)
            hi = jnp.minimum(o_e, st + CH)
            acc_rows(lo - std, hi - lo)
            return 0

        lax.fori_loop(0, nch, chunk_body, 0)

        denom = jnp.maximum(cnt.astype(jnp.float32), 1.0)
        for g in range(NG):
            sm = accbuf[pl.ds(2 * D + g * 16, 16)]
            accbuf[pl.ds(2 * D + g * 16, 16)] = sm / denom

        pltpu.sync_copy(accbuf, out_hbm.at[seg])
        return 0

    lax.fori_loop(0, SEG_PER_W, seg_body, 0)


def _pool(x2d, batch):
    mesh = plsc.VectorSubcoreMesh(core_axis_name="c", subcore_axis_name="s")
    call = functools.partial(
        pl.kernel,
        mesh=mesh,
        out_type=jax.ShapeDtypeStruct((NSEG, 3 * D), jnp.float32),
        scratch_types=[
            pltpu.VMEM((N + 16,), jnp.int32),
            pltpu.VMEM((CH, D), jnp.float32),
            pltpu.VMEM((3 * D,), jnp.float32),
            pltpu.SMEM((SEG_PER_W + 1,), jnp.int32),
            pltpu.SemaphoreType.DMA,
        ],
    )(_pool_kernel)
    return call(x2d, batch)


def _mm_kernel(feat_ref, w_ref, b_ref, out_ref):
    out_ref[...] = (
        jnp.dot(feat_ref[...], w_ref[...], preferred_element_type=jnp.float32)
        + b_ref[...]
    )


def _mm(feat, W, b):
    return pl.pallas_call(
        _mm_kernel,
        out_shape=jax.ShapeDtypeStruct((NSEG, D), jnp.float32),
    )(feat, W, b.reshape(1, D))


def kernel(x, batch, W, b):
    feat = _pool(x, batch.astype(jnp.int32))
    return _mm(feat, W, b)
